# baseline (speedup 1.0000x reference)
import functools

import jax
import jax.numpy as jnp
from jax import lax
from jax.experimental import pallas as pl
from jax.experimental.pallas import tpu as pltpu
from jax.experimental.pallas import tpu_sc as plsc

_LANES = 16
_TILES = 16
_NUM_SEGMENTS = 512
_BLK = _NUM_SEGMENTS // _TILES
_SUBACCS = 8
_STRIDE = _NUM_SEGMENTS + 3
_UNROLL = 8


@functools.lru_cache(maxsize=None)
def _make_seg_sum(nvec_total: int):
    base_vecs = nvec_total // _TILES
    extra = nvec_total % _TILES
    max_vecs = base_vecs + (1 if extra else 0)
    acc8_words = -(-_STRIDE * _SUBACCS // _LANES) * _LANES
    mesh = plsc.VectorSubcoreMesh(
        core_axis_name="c", subcore_axis_name="s", num_cores=1
    )

    @functools.partial(
        pl.kernel,
        out_type=jax.ShapeDtypeStruct((_NUM_SEGMENTS,), jnp.float32),
        mesh=mesh,
        compiler_params=pltpu.CompilerParams(
            needs_layout_passes=False,
            disable_bounds_checks=True,
            disable_semaphore_checks=True,
            use_tc_tiling_on_sc=False,
        ),
        scratch_types=[
            pltpu.VMEM((max_vecs * _LANES,), jnp.float32),
            pltpu.VMEM((max_vecs * _LANES,), jnp.int32),
            pltpu.VMEM((acc8_words,), jnp.float32),
            pltpu.VMEM((_NUM_SEGMENTS,), jnp.float32),
            pltpu.VMEM((_TILES, _BLK), jnp.float32),
            pltpu.VMEM((_BLK,), jnp.float32),
            pltpu.VMEM_SHARED((_TILES, _NUM_SEGMENTS), jnp.float32),
            pltpu.SemaphoreType.DMA,
            pltpu.SemaphoreType.DMA,
        ],
    )
    def seg_sum(val_hbm, idx_hbm, out_hbm, val_v, idx_v, acc8_v, acc_v,
                colbuf_v, res_v, shared, sem0, sem1):
        wid = lax.axis_index("s")
        base = (wid * base_vecs + jnp.minimum(wid, extra)) * _LANES

        cnt_max = max_vecs * _LANES
        half = (max_vecs // 2) * _LANES
        win = jnp.minimum(base, nvec_total * _LANES - cnt_max)
        delta = base - win
        cp0 = pltpu.async_copy(val_hbm.at[pl.ds(win, half)],
                               val_v.at[pl.ds(0, half)], sem0)
        cp1 = pltpu.async_copy(idx_hbm.at[pl.ds(win, half)],
                               idx_v.at[pl.ds(0, half)], sem0)
        rest = cnt_max - half
        cp2 = pltpu.async_copy(val_hbm.at[pl.ds(win + half, rest)],
                               val_v.at[pl.ds(half, rest)], sem1)
        cp3 = pltpu.async_copy(idx_hbm.at[pl.ds(win + half, rest)],
                               idx_v.at[pl.ds(half, rest)], sem1)

        zeros16 = jnp.zeros((_LANES,), jnp.float32)
        iota16 = lax.iota(jnp.int32, _LANES)
        lane_off = (iota16 & (_SUBACCS - 1)) * _STRIDE

        @plsc.parallel_loop(0, acc8_words, step=_LANES, unroll=8)
        def _zero(j):
            acc8_v[pl.ds(pl.multiple_of(j, _LANES), _LANES)] = zeros16

        def scat(off):
            off = pl.multiple_of(off, _LANES)
            v = val_v[pl.ds(off, _LANES)]
            b = idx_v[pl.ds(off, _LANES)]
            plsc.addupdate_scatter(acc8_v, [b + lane_off], v)

        cp0.wait()
        cp1.wait()

        @plsc.parallel_loop(0, half - _LANES, step=_LANES, unroll=_UNROLL)
        def _scatter_lo(i):
            scat(i + delta)

        cp2.wait()
        cp3.wait()

        @plsc.parallel_loop(half - _LANES, base_vecs * _LANES, step=_LANES,
                            unroll=_UNROLL)
        def _scatter_hi(i):
            scat(i + delta)

        if extra:
            pl.when(wid < extra)(
                functools.partial(scat, base_vecs * _LANES + delta))

        @plsc.parallel_loop(0, _NUM_SEGMENTS, step=_LANES, unroll=2)
        def _fold(j):
            seg = pl.multiple_of(j, _LANES) + iota16
            s = plsc.load_gather(acc8_v, [seg])
            for l in range(1, _SUBACCS):
                s = s + plsc.load_gather(acc8_v, [seg + l * _STRIDE])
            acc_v[pl.ds(pl.multiple_of(j, _LANES), _LANES)] = s

        pltpu.sync_copy(acc_v, shared.at[wid])
        plsc.subcore_barrier()

        col = pl.multiple_of(wid * _BLK, _BLK)
        cps = [pltpu.async_copy(shared.at[r, pl.ds(col, _BLK)],
                                colbuf_v.at[r], sem0)
               for r in range(_TILES)]
        for cp in cps:
            cp.wait()

        lo = [colbuf_v[r, pl.ds(0, _LANES)] for r in range(_TILES)]
        hi = [colbuf_v[r, pl.ds(_LANES, _LANES)] for r in range(_TILES)]
        while len(lo) > 1:
            lo = [lo[k] + lo[k + 1] for k in range(0, len(lo), 2)]
            hi = [hi[k] + hi[k + 1] for k in range(0, len(hi), 2)]
        res_v[pl.ds(0, _LANES)] = lo[0]
        res_v[pl.ds(_LANES, _LANES)] = hi[0]
        pltpu.sync_copy(res_v, out_hbm.at[pl.ds(col, _BLK)])

    return seg_sum


def kernel(atomic_energy, batch):
    n = atomic_energy.shape[0]
    src = jnp.squeeze(atomic_energy, axis=1)
    rem = n % _LANES
    if rem:
        pad = _LANES - rem
        src = jnp.pad(src, (0, pad))
        batch = jnp.pad(batch, (0, pad), constant_values=_NUM_SEGMENTS - 1)
        n += pad
    return _make_seg_sum(n // _LANES)(src, batch)


# speedup vs baseline: 1.0177x; 1.0177x over previous
import functools

import jax
import jax.numpy as jnp
from jax import lax
from jax.experimental import pallas as pl
from jax.experimental.pallas import tpu as pltpu
from jax.experimental.pallas import tpu_sc as plsc

_LANES = 16
_TILES = 16
_NUM_SEGMENTS = 512
_BLK = _NUM_SEGMENTS // _TILES
_SUBACCS = 8
_STRIDE = _NUM_SEGMENTS + 3
_UNROLL = 4


@functools.lru_cache(maxsize=None)
def _make_seg_sum(nvec_total: int):
    base_vecs = nvec_total // _TILES
    extra = nvec_total % _TILES
    max_vecs = base_vecs + (1 if extra else 0)
    acc8_words = -(-_STRIDE * _SUBACCS // _LANES) * _LANES
    mesh = plsc.VectorSubcoreMesh(
        core_axis_name="c", subcore_axis_name="s", num_cores=1
    )

    @functools.partial(
        pl.kernel,
        out_type=jax.ShapeDtypeStruct((_NUM_SEGMENTS,), jnp.float32),
        mesh=mesh,
        compiler_params=pltpu.CompilerParams(
            needs_layout_passes=False,
            disable_bounds_checks=True,
            disable_semaphore_checks=True,
            use_tc_tiling_on_sc=False,
        ),
        scratch_types=[
            pltpu.VMEM((max_vecs * _LANES,), jnp.float32),
            pltpu.VMEM((max_vecs * _LANES,), jnp.int32),
            pltpu.VMEM((acc8_words,), jnp.float32),
            pltpu.VMEM((_NUM_SEGMENTS,), jnp.float32),
            pltpu.VMEM((_TILES, _BLK), jnp.float32),
            pltpu.VMEM((_BLK,), jnp.float32),
            pltpu.VMEM_SHARED((_TILES, _NUM_SEGMENTS), jnp.float32),
            pltpu.SemaphoreType.DMA,
            pltpu.SemaphoreType.DMA,
        ],
    )
    def seg_sum(val_hbm, idx_hbm, out_hbm, val_v, idx_v, acc8_v, acc_v,
                colbuf_v, res_v, shared, sem0, sem1):
        wid = lax.axis_index("s")
        base = (wid * base_vecs + jnp.minimum(wid, extra)) * _LANES

        cnt_max = max_vecs * _LANES
        half = (max_vecs // 2) * _LANES
        win = jnp.minimum(base, nvec_total * _LANES - cnt_max)
        delta = base - win
        cp0 = pltpu.async_copy(val_hbm.at[pl.ds(win, half)],
                               val_v.at[pl.ds(0, half)], sem0)
        cp1 = pltpu.async_copy(idx_hbm.at[pl.ds(win, half)],
                               idx_v.at[pl.ds(0, half)], sem0)
        rest = cnt_max - half
        cp2 = pltpu.async_copy(val_hbm.at[pl.ds(win + half, rest)],
                               val_v.at[pl.ds(half, rest)], sem1)
        cp3 = pltpu.async_copy(idx_hbm.at[pl.ds(win + half, rest)],
                               idx_v.at[pl.ds(half, rest)], sem1)

        zeros16 = jnp.zeros((_LANES,), jnp.float32)
        iota16 = lax.iota(jnp.int32, _LANES)
        lane_off = (iota16 & (_SUBACCS - 1)) * _STRIDE

        @plsc.parallel_loop(0, acc8_words, step=_LANES, unroll=4)
        def _zero(j):
            acc8_v[pl.ds(pl.multiple_of(j, _LANES), _LANES)] = zeros16

        def scat(off):
            off = pl.multiple_of(off, _LANES)
            v = val_v[pl.ds(off, _LANES)]
            b = idx_v[pl.ds(off, _LANES)]
            plsc.addupdate_scatter(acc8_v, [b + lane_off], v)

        cp0.wait()
        cp1.wait()

        @plsc.parallel_loop(0, half - _LANES, step=_LANES, unroll=_UNROLL)
        def _scatter_lo(i):
            scat(i + delta)

        cp2.wait()
        cp3.wait()

        @plsc.parallel_loop(half - _LANES, base_vecs * _LANES, step=_LANES,
                            unroll=_UNROLL)
        def _scatter_hi(i):
            scat(i + delta)

        if extra:
            pl.when(wid < extra)(
                functools.partial(scat, base_vecs * _LANES + delta))

        @plsc.parallel_loop(0, _NUM_SEGMENTS, step=_LANES, unroll=1)
        def _fold(j):
            seg = pl.multiple_of(j, _LANES) + iota16
            s = plsc.load_gather(acc8_v, [seg])
            for l in range(1, _SUBACCS):
                s = s + plsc.load_gather(acc8_v, [seg + l * _STRIDE])
            acc_v[pl.ds(pl.multiple_of(j, _LANES), _LANES)] = s

        pltpu.sync_copy(acc_v, shared.at[wid])
        plsc.subcore_barrier()

        col = pl.multiple_of(wid * _BLK, _BLK)
        cps = [pltpu.async_copy(shared.at[r, pl.ds(col, _BLK)],
                                colbuf_v.at[r], sem0)
               for r in range(_TILES)]
        for cp in cps:
            cp.wait()

        lo = [colbuf_v[r, pl.ds(0, _LANES)] for r in range(_TILES)]
        hi = [colbuf_v[r, pl.ds(_LANES, _LANES)] for r in range(_TILES)]
        while len(lo) > 1:
            lo = [lo[k] + lo[k + 1] for k in range(0, len(lo), 2)]
            hi = [hi[k] + hi[k + 1] for k in range(0, len(hi), 2)]
        res_v[pl.ds(0, _LANES)] = lo[0]
        res_v[pl.ds(_LANES, _LANES)] = hi[0]
        pltpu.sync_copy(res_v, out_hbm.at[pl.ds(col, _BLK)])

    return seg_sum


def kernel(atomic_energy, batch):
    n = atomic_energy.shape[0]
    src = jnp.squeeze(atomic_energy, axis=1)
    rem = n % _LANES
    if rem:
        pad = _LANES - rem
        src = jnp.pad(src, (0, pad))
        batch = jnp.pad(batch, (0, pad), constant_values=_NUM_SEGMENTS - 1)
        n += pad
    return _make_seg_sum(n // _LANES)(src, batch)


# speedup vs baseline: 1.0278x; 1.0099x over previous
import functools

import jax
import jax.numpy as jnp
from jax import lax
from jax.experimental import pallas as pl
from jax.experimental.pallas import tpu as pltpu
from jax.experimental.pallas import tpu_sc as plsc

_LANES = 16
_TILES = 16
_NUM_SEGMENTS = 512
_BLK = _NUM_SEGMENTS // _TILES
_SUBACCS = 8
_STRIDE = _NUM_SEGMENTS + 3
_UNROLL = 4


@functools.lru_cache(maxsize=None)
def _make_seg_sum(nvec_total: int):
    base_vecs = nvec_total // _TILES
    extra = nvec_total % _TILES
    max_vecs = base_vecs + (1 if extra else 0)
    acc8_words = -(-_STRIDE * _SUBACCS // _LANES) * _LANES
    mesh = plsc.VectorSubcoreMesh(
        core_axis_name="c", subcore_axis_name="s", num_cores=1
    )

    @functools.partial(
        pl.kernel,
        out_type=jax.ShapeDtypeStruct((_NUM_SEGMENTS,), jnp.float32),
        mesh=mesh,
        compiler_params=pltpu.CompilerParams(
            needs_layout_passes=False,
            disable_bounds_checks=True,
            disable_semaphore_checks=True,
            use_tc_tiling_on_sc=False,
        ),
        scratch_types=[
            pltpu.VMEM((max_vecs * _LANES,), jnp.float32),
            pltpu.VMEM((max_vecs * _LANES,), jnp.int32),
            pltpu.VMEM((acc8_words,), jnp.float32),
            pltpu.VMEM((_NUM_SEGMENTS,), jnp.float32),
            pltpu.VMEM((_TILES, _BLK), jnp.float32),
            pltpu.VMEM((_BLK,), jnp.float32),
            pltpu.VMEM_SHARED((_TILES, _NUM_SEGMENTS), jnp.float32),
            pltpu.SemaphoreType.DMA,
            pltpu.SemaphoreType.DMA,
        ],
    )
    def seg_sum(val_hbm, idx_hbm, out_hbm, val_v, idx_v, acc8_v, acc_v,
                colbuf_v, res_v, shared, sem0, sem1):
        wid = lax.axis_index("s")
        base = (wid * base_vecs + jnp.minimum(wid, extra)) * _LANES

        cnt_max = max_vecs * _LANES
        half = (max_vecs // 2) * _LANES
        win = jnp.minimum(base, nvec_total * _LANES - cnt_max)
        delta = base - win
        cp0 = pltpu.async_copy(val_hbm.at[pl.ds(win, half)],
                               val_v.at[pl.ds(0, half)], sem0)
        cp1 = pltpu.async_copy(idx_hbm.at[pl.ds(win, half)],
                               idx_v.at[pl.ds(0, half)], sem0)
        rest = cnt_max - half
        cp2 = pltpu.async_copy(val_hbm.at[pl.ds(win + half, rest)],
                               val_v.at[pl.ds(half, rest)], sem1)
        cp3 = pltpu.async_copy(idx_hbm.at[pl.ds(win + half, rest)],
                               idx_v.at[pl.ds(half, rest)], sem1)

        zeros16 = jnp.zeros((_LANES,), jnp.float32)
        iota16 = lax.iota(jnp.int32, _LANES)
        lane_off = (iota16 & (_SUBACCS - 1)) * _STRIDE

        @plsc.parallel_loop(0, acc8_words, step=_LANES, unroll=4)
        def _zero(j):
            acc8_v[pl.ds(pl.multiple_of(j, _LANES), _LANES)] = zeros16

        def scat(off):
            off = pl.multiple_of(off, _LANES)
            v = val_v[pl.ds(off, _LANES)]
            b = idx_v[pl.ds(off, _LANES)]
            plsc.addupdate_scatter(acc8_v, [b + lane_off], v)

        cp0.wait()
        cp1.wait()

        @plsc.parallel_loop(0, half - _LANES, step=_LANES, unroll=_UNROLL)
        def _scatter_lo(i):
            scat(i + delta)

        cp2.wait()
        cp3.wait()

        @plsc.parallel_loop(half - _LANES, base_vecs * _LANES, step=_LANES,
                            unroll=_UNROLL)
        def _scatter_hi(i):
            scat(i + delta)

        if extra:
            pl.when(wid < extra)(
                functools.partial(scat, base_vecs * _LANES + delta))

        @plsc.parallel_loop(0, _NUM_SEGMENTS, step=_LANES, unroll=1)
        def _fold(j):
            seg = pl.multiple_of(j, _LANES) + iota16
            s = plsc.load_gather(acc8_v, [seg])
            for l in range(1, _SUBACCS):
                s = s + plsc.load_gather(acc8_v, [seg + l * _STRIDE])
            acc_v[pl.ds(pl.multiple_of(j, _LANES), _LANES)] = s

        pltpu.sync_copy(acc_v, shared.at[wid])
        plsc.subcore_barrier()

        col = pl.multiple_of(wid * _BLK, _BLK)
        pltpu.sync_copy(shared.at[:, pl.ds(col, _BLK)], colbuf_v)

        lo = [colbuf_v[r, pl.ds(0, _LANES)] for r in range(_TILES)]
        hi = [colbuf_v[r, pl.ds(_LANES, _LANES)] for r in range(_TILES)]
        while len(lo) > 1:
            lo = [lo[k] + lo[k + 1] for k in range(0, len(lo), 2)]
            hi = [hi[k] + hi[k + 1] for k in range(0, len(hi), 2)]
        res_v[pl.ds(0, _LANES)] = lo[0]
        res_v[pl.ds(_LANES, _LANES)] = hi[0]
        pltpu.sync_copy(res_v, out_hbm.at[pl.ds(col, _BLK)])

    return seg_sum


def kernel(atomic_energy, batch):
    n = atomic_energy.shape[0]
    src = jnp.squeeze(atomic_energy, axis=1)
    rem = n % _LANES
    if rem:
        pad = _LANES - rem
        src = jnp.pad(src, (0, pad))
        batch = jnp.pad(batch, (0, pad), constant_values=_NUM_SEGMENTS - 1)
        n += pad
    return _make_seg_sum(n // _LANES)(src, batch)
